# baseline (speedup 1.0000x reference)
import functools

import jax
import jax.numpy as jnp
from jax import lax
from jax.experimental import pallas as pl
from jax.experimental.pallas import tpu as pltpu
from jax.experimental.pallas import tpu_sc as plsc

BATCH = 4096
INPUT_DIM = 39
N_CONT = 13
N_CAT = 26
VOCAB = 100000
EMB = 32
EMB2 = EMB // 2
LANES = 16
OUT_DIM = N_CONT + N_CAT * EMB

NUM_CORES = 2
NUM_SUBCORES = 16
NUM_WORKERS = NUM_CORES * NUM_SUBCORES
B_PER_W = BATCH // NUM_WORKERS
VREGS_PER_TABLE = B_PER_W // LANES
DESC_PER_TABLE = EMB2 * VREGS_PER_TABLE

_mesh = plsc.VectorSubcoreMesh(
    core_axis_name="c", subcore_axis_name="s",
    num_cores=NUM_CORES, num_subcores=NUM_SUBCORES,
)


@functools.partial(
    pl.kernel,
    out_type=jax.ShapeDtypeStruct((OUT_DIM, BATCH), jnp.float32),
    mesh=_mesh,
    compiler_params=pltpu.CompilerParams(
        use_tc_tiling_on_sc=False, needs_layout_passes=False),
    scratch_types=[
        pltpu.VMEM((N_CAT, B_PER_W), jnp.float32),
        pltpu.VMEM((OUT_DIM, B_PER_W), jnp.float32),
        pltpu.VMEM((DESC_PER_TABLE * LANES,), jnp.int32),
        pltpu.SemaphoreType.DMA,
    ],
)
def _emb_kernel(tt_hbm, xt_hbm, out_hbm, xi_v, asm_v, pair_v, sem):
    wid = lax.axis_index("s") * NUM_CORES + lax.axis_index("c")
    base_b = wid * B_PER_W

    pltpu.sync_copy(
        xt_hbm.at[pl.ds(N_CONT, N_CAT), pl.ds(base_b, B_PER_W)], xi_v)
    pltpu.sync_copy(xt_hbm.at[pl.ds(0, N_CONT), pl.ds(base_b, B_PER_W)],
                    asm_v.at[pl.ds(0, N_CONT), :])

    himask = jnp.full((LANES,), -65536, jnp.int32)

    def per_table(j, carry):
        vjs = [
            xi_v[j, pl.ds(h * LANES, LANES)].astype(jnp.int32)
            for h in range(VREGS_PER_TABLE)
        ]
        for e2 in range(EMB2):
            base = (j * EMB2 + e2) * VOCAB
            for h in range(VREGS_PER_TABLE):
                pltpu.async_copy(
                    tt_hbm.at[vjs[h] + base],
                    pair_v.at[pl.ds((e2 * VREGS_PER_TABLE + h) * LANES,
                                    LANES)],
                    sem)
        pltpu.make_async_copy(
            tt_hbm.at[pl.ds(0, DESC_PER_TABLE * LANES)], pair_v, sem).wait()
        row0 = N_CONT + j * EMB
        for e2 in range(EMB2):
            for h in range(VREGS_PER_TABLE):
                px = pair_v[pl.ds((e2 * VREGS_PER_TABLE + h) * LANES, LANES)]
                lo = plsc.bitcast(px << 16, jnp.float32)
                hi = plsc.bitcast(px & himask, jnp.float32)
                asm_v[row0 + 2 * e2, pl.ds(h * LANES, LANES)] = lo
                asm_v[row0 + 2 * e2 + 1, pl.ds(h * LANES, LANES)] = hi
        return carry

    lax.fori_loop(0, N_CAT, per_table, 0)

    pltpu.sync_copy(asm_v, out_hbm.at[:, pl.ds(base_b, B_PER_W)])


def kernel(x, tables):
    xt = x.T
    pairs = jax.lax.bitcast_convert_type(
        tables.astype(jnp.bfloat16).reshape(N_CAT, VOCAB, EMB2, 2),
        jnp.int32)
    tt = jnp.transpose(pairs, (0, 2, 1)).reshape(N_CAT * EMB2 * VOCAB)
    out_t = _emb_kernel(tt, xt)
    return out_t.T


# speedup vs baseline: 2.6676x; 2.6676x over previous
import functools

import jax
import jax.numpy as jnp
from jax import lax
from jax.experimental import pallas as pl
from jax.experimental.pallas import tpu as pltpu
from jax.experimental.pallas import tpu_sc as plsc

BATCH = 4096
INPUT_DIM = 39
N_CONT = 13
N_CAT = 26
VOCAB = 100000
EMB = 32
LANES = 16
OUT_DIM = N_CONT + N_CAT * EMB

NUM_CORES = 2
NUM_SUBCORES = 16
NUM_WORKERS = NUM_CORES * NUM_SUBCORES
B_PER_W = BATCH // NUM_WORKERS
VREGS_PER_TABLE = B_PER_W // LANES

_mesh = plsc.VectorSubcoreMesh(
    core_axis_name="c", subcore_axis_name="s",
    num_cores=NUM_CORES, num_subcores=NUM_SUBCORES,
)


@functools.partial(
    pl.kernel,
    out_type=jax.ShapeDtypeStruct((OUT_DIM, BATCH), jnp.float32),
    mesh=_mesh,
    compiler_params=pltpu.CompilerParams(
        use_tc_tiling_on_sc=False, needs_layout_passes=False),
    scratch_types=[
        pltpu.VMEM((N_CAT, B_PER_W), jnp.float32),
        pltpu.VMEM((OUT_DIM, B_PER_W), jnp.float32),
        pltpu.SemaphoreType.DMA,
    ],
)
def _emb_kernel(tt_hbm, xt_hbm, out_hbm, xi_v, asm_v, sem):
    wid = lax.axis_index("s") * NUM_CORES + lax.axis_index("c")
    base_b = wid * B_PER_W

    pltpu.sync_copy(
        xt_hbm.at[pl.ds(N_CONT, N_CAT), pl.ds(base_b, B_PER_W)], xi_v)
    pltpu.sync_copy(xt_hbm.at[pl.ds(0, N_CONT), pl.ds(base_b, B_PER_W)],
                    asm_v.at[pl.ds(0, N_CONT), :])

    def table_bytes_wait():
        pltpu.make_async_copy(
            xt_hbm.at[pl.ds(0, EMB), pl.ds(0, B_PER_W)],
            asm_v.at[pl.ds(N_CONT, EMB), :],
            sem).wait()

    def per_table(j, carry):
        vjs = [
            xi_v[j, pl.ds(h * LANES, LANES)].astype(jnp.int32)
            for h in range(VREGS_PER_TABLE)
        ]
        row0 = N_CONT + j * EMB
        for e in range(EMB):
            base = (j * EMB + e) * VOCAB
            for h in range(VREGS_PER_TABLE):
                flat_idx = vjs[h] + base
                pltpu.async_copy(
                    tt_hbm.at[flat_idx],
                    asm_v.at[row0 + e, pl.ds(h * LANES, LANES)],
                    sem)

        @pl.when(j > 0)
        def _():
            table_bytes_wait()

        return carry

    lax.fori_loop(0, N_CAT, per_table, 0)
    table_bytes_wait()

    pltpu.sync_copy(asm_v, out_hbm.at[:, pl.ds(base_b, B_PER_W)])


def kernel(x, tables):
    xt = x.T
    tt = jnp.transpose(tables, (0, 2, 1)).reshape(N_CAT * EMB * VOCAB)
    out_t = _emb_kernel(tt, xt)
    return out_t.T


# speedup vs baseline: 2.6863x; 1.0070x over previous
import functools

import jax
import jax.numpy as jnp
from jax import lax
from jax.experimental import pallas as pl
from jax.experimental.pallas import tpu as pltpu
from jax.experimental.pallas import tpu_sc as plsc

BATCH = 4096
INPUT_DIM = 39
N_CONT = 13
N_CAT = 26
VOCAB = 100000
EMB = 32
LANES = 16
OUT_DIM = N_CONT + N_CAT * EMB

NUM_CORES = 2
NUM_SUBCORES = 16
NUM_WORKERS = NUM_CORES * NUM_SUBCORES
B_PER_W = BATCH // NUM_WORKERS
VREGS_PER_TABLE = B_PER_W // LANES

_mesh = plsc.VectorSubcoreMesh(
    core_axis_name="c", subcore_axis_name="s",
    num_cores=NUM_CORES, num_subcores=NUM_SUBCORES,
)


@functools.partial(
    pl.kernel,
    out_type=jax.ShapeDtypeStruct((OUT_DIM, BATCH), jnp.float32),
    mesh=_mesh,
    compiler_params=pltpu.CompilerParams(
        use_tc_tiling_on_sc=False, needs_layout_passes=False),
    scratch_types=[
        pltpu.VMEM((N_CAT, B_PER_W), jnp.float32),
        pltpu.VMEM((OUT_DIM, B_PER_W), jnp.float32),
        pltpu.SemaphoreType.DMA,
    ],
)
def _emb_kernel(tt_hbm, xt_hbm, out_hbm, xi_v, asm_v, sem):
    wid = lax.axis_index("s") * NUM_CORES + lax.axis_index("c")
    base_b = wid * B_PER_W

    pltpu.sync_copy(
        xt_hbm.at[pl.ds(N_CONT, N_CAT), pl.ds(base_b, B_PER_W)], xi_v)
    pltpu.sync_copy(xt_hbm.at[pl.ds(0, N_CONT), pl.ds(base_b, B_PER_W)],
                    asm_v.at[pl.ds(0, N_CONT), :])

    def table_bytes_wait():
        pltpu.make_async_copy(
            xt_hbm.at[pl.ds(0, EMB), pl.ds(0, B_PER_W)],
            asm_v.at[pl.ds(N_CONT, EMB), :],
            sem).wait()

    def per_table(j, carry):
        vjs = [
            xi_v[j, pl.ds(h * LANES, LANES)].astype(jnp.int32)
            for h in range(VREGS_PER_TABLE)
        ]
        row0 = N_CONT + j * EMB
        for e in range(EMB):
            base = (j * EMB + e) * VOCAB
            for h in range(VREGS_PER_TABLE):
                flat_idx = vjs[h] + base
                pltpu.async_copy(
                    tt_hbm.at[flat_idx],
                    asm_v.at[row0 + e, pl.ds(h * LANES, LANES)],
                    sem)

        @pl.when(j > 1)
        def _():
            table_bytes_wait()

        return carry

    lax.fori_loop(0, N_CAT, per_table, 0)
    table_bytes_wait()
    table_bytes_wait()

    pltpu.sync_copy(asm_v, out_hbm.at[:, pl.ds(base_b, B_PER_W)])


def kernel(x, tables):
    xt = x.T
    tt = jnp.transpose(tables, (0, 2, 1)).reshape(N_CAT * EMB * VOCAB)
    out_t = _emb_kernel(tt, xt)
    return out_t.T


# speedup vs baseline: 2.7146x; 1.0105x over previous
import functools

import jax
import jax.numpy as jnp
from jax import lax
from jax.experimental import pallas as pl
from jax.experimental.pallas import tpu as pltpu
from jax.experimental.pallas import tpu_sc as plsc

BATCH = 4096
INPUT_DIM = 39
N_CONT = 13
N_CAT = 26
VOCAB = 100000
EMB = 32
LANES = 16
OUT_DIM = N_CONT + N_CAT * EMB

NUM_CORES = 2
NUM_SUBCORES = 16
NUM_WORKERS = NUM_CORES * NUM_SUBCORES
B_PER_W = BATCH // NUM_WORKERS
VREGS_PER_TABLE = B_PER_W // LANES

_mesh = plsc.VectorSubcoreMesh(
    core_axis_name="c", subcore_axis_name="s",
    num_cores=NUM_CORES, num_subcores=NUM_SUBCORES,
)


@functools.partial(
    pl.kernel,
    out_type=jax.ShapeDtypeStruct((OUT_DIM, BATCH), jnp.float32),
    mesh=_mesh,
    compiler_params=pltpu.CompilerParams(
        use_tc_tiling_on_sc=False, needs_layout_passes=False),
    scratch_types=[
        pltpu.VMEM((N_CAT, B_PER_W), jnp.float32),
        pltpu.VMEM((OUT_DIM, B_PER_W), jnp.float32),
        pltpu.SemaphoreType.DMA,
    ],
)
def _emb_kernel(tt_hbm, xt_hbm, out_hbm, xi_v, asm_v, sem):
    wid = lax.axis_index("s") * NUM_CORES + lax.axis_index("c")
    base_b = wid * B_PER_W

    pltpu.sync_copy(
        xt_hbm.at[pl.ds(N_CONT, N_CAT), pl.ds(base_b, B_PER_W)], xi_v)
    pltpu.sync_copy(xt_hbm.at[pl.ds(0, N_CONT), pl.ds(base_b, B_PER_W)],
                    asm_v.at[pl.ds(0, N_CONT), :])

    def table_bytes_wait():
        pltpu.make_async_copy(
            xt_hbm.at[pl.ds(0, EMB), pl.ds(0, B_PER_W)],
            asm_v.at[pl.ds(N_CONT, EMB), :],
            sem).wait()

    def per_table(j, carry):
        vjs = [
            xi_v[j, pl.ds(h * LANES, LANES)].astype(jnp.int32)
            for h in range(VREGS_PER_TABLE)
        ]
        row0 = N_CONT + j * EMB
        for e in range(EMB):
            base = (j * EMB + e) * VOCAB
            for h in range(VREGS_PER_TABLE):
                flat_idx = vjs[h] + base
                pltpu.async_copy(
                    tt_hbm.at[flat_idx],
                    asm_v.at[row0 + e, pl.ds(h * LANES, LANES)],
                    sem)

        @pl.when(j > 3)
        def _():
            table_bytes_wait()

        return carry

    lax.fori_loop(0, N_CAT, per_table, 0)
    for _ in range(4):
        table_bytes_wait()

    pltpu.sync_copy(asm_v, out_hbm.at[:, pl.ds(base_b, B_PER_W)])


def kernel(x, tables):
    xt = x.T
    tt = jnp.transpose(tables, (0, 2, 1)).reshape(N_CAT * EMB * VOCAB)
    out_t = _emb_kernel(tt, xt)
    return out_t.T


# speedup vs baseline: 2.7314x; 1.0062x over previous
import functools

import jax
import jax.numpy as jnp
from jax import lax
from jax.experimental import pallas as pl
from jax.experimental.pallas import tpu as pltpu
from jax.experimental.pallas import tpu_sc as plsc

BATCH = 4096
INPUT_DIM = 39
N_CONT = 13
N_CAT = 26
VOCAB = 100000
EMB = 32
LANES = 16
OUT_DIM = N_CONT + N_CAT * EMB

NUM_CORES = 2
NUM_SUBCORES = 16
NUM_WORKERS = NUM_CORES * NUM_SUBCORES
B_PER_W = BATCH // NUM_WORKERS
VREGS_PER_TABLE = B_PER_W // LANES

_mesh = plsc.VectorSubcoreMesh(
    core_axis_name="c", subcore_axis_name="s",
    num_cores=NUM_CORES, num_subcores=NUM_SUBCORES,
)


@functools.partial(
    pl.kernel,
    out_type=jax.ShapeDtypeStruct((OUT_DIM, BATCH), jnp.float32),
    mesh=_mesh,
    compiler_params=pltpu.CompilerParams(
        use_tc_tiling_on_sc=False, needs_layout_passes=False),
    scratch_types=[
        pltpu.VMEM((N_CAT, B_PER_W), jnp.float32),
        pltpu.VMEM((OUT_DIM, B_PER_W), jnp.float32),
        pltpu.SemaphoreType.DMA,
    ],
)
def _emb_kernel(tt_hbm, xt_hbm, out_hbm, xi_v, asm_v, sem):
    wid = lax.axis_index("s") * NUM_CORES + lax.axis_index("c")
    base_b = wid * B_PER_W

    pltpu.sync_copy(
        xt_hbm.at[pl.ds(N_CONT, N_CAT), pl.ds(base_b, B_PER_W)], xi_v)
    pltpu.sync_copy(xt_hbm.at[pl.ds(0, N_CONT), pl.ds(base_b, B_PER_W)],
                    asm_v.at[pl.ds(0, N_CONT), :])

    def table_bytes_wait():
        pltpu.make_async_copy(
            xt_hbm.at[pl.ds(0, EMB), pl.ds(0, B_PER_W)],
            asm_v.at[pl.ds(N_CONT, EMB), :],
            sem).wait()

    def per_table(j, carry):
        vjs = [
            xi_v[j, pl.ds(h * LANES, LANES)].astype(jnp.int32)
            for h in range(VREGS_PER_TABLE)
        ]
        row0 = N_CONT + j * EMB
        for e in range(EMB):
            base = (j * EMB + e) * VOCAB
            for h in range(VREGS_PER_TABLE):
                flat_idx = vjs[h] + base
                pltpu.async_copy(
                    tt_hbm.at[flat_idx],
                    asm_v.at[row0 + e, pl.ds(h * LANES, LANES)],
                    sem)

        @pl.when(j > 7)
        def _():
            table_bytes_wait()

        return carry

    lax.fori_loop(0, N_CAT, per_table, 0)
    for _ in range(8):
        table_bytes_wait()

    pltpu.sync_copy(asm_v, out_hbm.at[:, pl.ds(base_b, B_PER_W)])


def kernel(x, tables):
    xt = x.T
    tt = jnp.transpose(tables, (0, 2, 1)).reshape(N_CAT * EMB * VOCAB)
    out_t = _emb_kernel(tt, xt)
    return out_t.T
